# baseline (speedup 1.0000x reference)
import functools

import jax
import jax.numpy as jnp
from jax import lax
from jax.experimental import pallas as pl
from jax.experimental.pallas import tpu as pltpu
from jax.experimental.pallas import tpu_sc as plsc

N = 10000
E = 320000
D = 128

NC = 2
NS = 16
NW = NC * NS
K = 128
EPT = 10240
CH = EPT // K
EPAD = NW * EPT
KS = 120
EPTS = 10200
CHS = EPTS // KS
EPADS = NW * EPTS
NP = 10240
RPS = NP // NS
ZR = 128
DW = 16

_IDCHUNKS = ((0, 120), (120, 120), (240, 120), (360, 120), (480, 120),
             (600, 40))

_mesh = plsc.VectorSubcoreMesh(core_axis_name="c", subcore_axis_name="s")


def _deg_body(dst2, ones_h, zeros_h, out, dall, onesv, zbuf, acc):
    c = lax.axis_index("c")
    s = lax.axis_index("s")
    wid = c * NS + s
    pltpu.sync_copy(zeros_h, zbuf)
    for j in range(RPS // ZR):
        pltpu.sync_copy(zbuf, acc.at[pl.ds(s * RPS + j * ZR, ZR)])
    pltpu.sync_copy(ones_h, onesv)
    pltpu.sync_copy(dst2.at[pl.ds(wid * CH, CH)], dall)
    plsc.subcore_barrier()

    def step(i, carry):
        pltpu.sync_copy(onesv, acc.at[dall.at[i]], add=True)
        return carry

    lax.fori_loop(0, CH, step, 0)
    plsc.subcore_barrier()
    for j in range(RPS // ZR):
        r0 = s * RPS + j * ZR
        pltpu.sync_copy(acc.at[pl.ds(r0, ZR)], zbuf)
        pltpu.sync_copy(zbuf, out.at[pl.ds(c * NP + r0, ZR)])


_deg = functools.partial(
    pl.kernel,
    out_type=jax.ShapeDtypeStruct((NC * NP, D), jnp.float32),
    mesh=_mesh,
    scratch_types=[
        pltpu.VMEM((CH, K), jnp.int32),
        pltpu.VMEM((K, D), jnp.float32),
        pltpu.VMEM((ZR, D), jnp.float32),
        pltpu.VMEM_SHARED((NP, D), jnp.float32),
    ],
)(_deg_body)


def _scatter_body(hs, src1, dst1, zeros_h, out,
                  si0, si1, si2, di0, di1, di2, r0, r1, r2, acc,
                  gi0, gi1, gi2, gg0, gg1, gg2):
    c = lax.axis_index("c")
    s = lax.axis_index("s")
    wid = c * NS + s
    sidx = (si0, si1, si2)
    didx = (di0, di1, di2)
    rows = (r0, r1, r2)
    semi = (gi0, gi1, gi2)
    semg = (gg0, gg1, gg2)
    pltpu.sync_copy(zeros_h.at[pl.ds(0, KS)], r0)
    for (o, sz) in _IDCHUNKS:
        pltpu.sync_copy(r0.at[pl.ds(0, sz)],
                        acc.at[pl.ds(s * RPS + o, sz)])
    plsc.subcore_barrier()

    base = wid * EPTS

    def load_idx(cc, b):
        off = pl.multiple_of(base + cc * KS, 8)
        pltpu.async_copy(src1.at[pl.ds(off, KS)], sidx[b], semi[b])
        pltpu.async_copy(dst1.at[pl.ds(off, KS)], didx[b], semi[b])

    def wait_idx(b):
        pltpu.make_async_copy(src1.at[pl.ds(0, KS)], sidx[b], semi[b]).wait()
        pltpu.make_async_copy(dst1.at[pl.ds(0, KS)], didx[b], semi[b]).wait()

    def gather(b):
        pltpu.async_copy(hs.at[sidx[b]], rows[b], semg[b])

    def wait_gather(b):
        pltpu.make_async_copy(hs.at[sidx[b]], rows[b], semg[b]).wait()

    for b in range(3):
        load_idx(b, b)
    wait_idx(0)
    gather(0)
    wait_idx(1)
    gather(1)

    def step(c3, carry):
        for b in range(3):
            cc = c3 * 3 + b
            bn = (b + 2) % 3

            @pl.when(cc + 2 < CHS)
            def _():
                wait_idx(bn)
                gather(bn)

            wait_gather(b)
            pltpu.sync_copy(rows[b], acc.at[didx[b]], add=True)

            @pl.when(cc + 3 < CHS)
            def _():
                load_idx(cc + 3, b)

        return carry

    lax.fori_loop(0, CHS // 3, step, 0)
    eb = (CHS - 1) % 3
    wait_gather(eb)
    pltpu.sync_copy(rows[eb], acc.at[didx[eb]], add=True)
    plsc.subcore_barrier()
    for (o, sz) in _IDCHUNKS:
        pltpu.sync_copy(acc.at[pl.ds(s * RPS + o, sz)], r0.at[pl.ds(0, sz)])
        pltpu.sync_copy(r0.at[pl.ds(0, sz)],
                        out.at[pl.ds(c * NP + s * RPS + o, sz)])


_scatter = functools.partial(
    pl.kernel,
    out_type=jax.ShapeDtypeStruct((NC * NP, D), jnp.float32),
    mesh=_mesh,
    scratch_types=[
        pltpu.VMEM((KS,), jnp.int32),
        pltpu.VMEM((KS,), jnp.int32),
        pltpu.VMEM((KS,), jnp.int32),
        pltpu.VMEM((KS,), jnp.int32),
        pltpu.VMEM((KS,), jnp.int32),
        pltpu.VMEM((KS,), jnp.int32),
        pltpu.VMEM((KS, D), jnp.float32),
        pltpu.VMEM((KS, D), jnp.float32),
        pltpu.VMEM((KS, D), jnp.float32),
        pltpu.VMEM_SHARED((NP, D), jnp.float32),
        pltpu.SemaphoreType.DMA,
        pltpu.SemaphoreType.DMA,
        pltpu.SemaphoreType.DMA,
        pltpu.SemaphoreType.DMA,
        pltpu.SemaphoreType.DMA,
        pltpu.SemaphoreType.DMA,
    ],
)(_scatter_body)


BR = 1000


def _tc1_body(dp_ref, x_ref, w_ref, hs_ref, dinv_ref):
    deg = 1.0 + dp_ref[0][:, :1] + dp_ref[1][:, :1]
    dinv = lax.rsqrt(deg)
    h = jnp.dot(x_ref[...], w_ref[...], preferred_element_type=jnp.float32)
    hs_ref[...] = h * dinv
    dinv_ref[...] = jnp.broadcast_to(dinv, (BR, DW))


def _tc1(degp, x, w):
    return pl.pallas_call(
        _tc1_body,
        grid=(N // BR,),
        in_specs=[
            pl.BlockSpec((NC, BR, D), lambda i: (0, i, 0)),
            pl.BlockSpec((BR, D), lambda i: (i, 0)),
            pl.BlockSpec((D, D), lambda i: (0, 0)),
        ],
        out_specs=[
            pl.BlockSpec((BR, D), lambda i: (i, 0)),
            pl.BlockSpec((BR, DW), lambda i: (i, 0)),
        ],
        out_shape=[
            jax.ShapeDtypeStruct((N, D), jnp.float32),
            jax.ShapeDtypeStruct((N, DW), jnp.float32),
        ],
    )(degp, x, w)


def _tc2_body(pp_ref, hs_ref, dinv_ref, b_ref, w_ref, out_ref):
    dinv = dinv_ref[...][:, :1]
    a = (pp_ref[0] + pp_ref[1] + hs_ref[...]) * dinv + b_ref[...]
    h = jnp.maximum(a, 0.0)
    out_ref[...] = jnp.dot(
        h, w_ref[...], preferred_element_type=jnp.float32) * dinv


def _tc2(pp, hs, dinv, b, w):
    return pl.pallas_call(
        _tc2_body,
        grid=(N // BR,),
        in_specs=[
            pl.BlockSpec((NC, BR, D), lambda i: (0, i, 0)),
            pl.BlockSpec((BR, D), lambda i: (i, 0)),
            pl.BlockSpec((BR, DW), lambda i: (i, 0)),
            pl.BlockSpec((1, D), lambda i: (0, 0)),
            pl.BlockSpec((D, D), lambda i: (0, 0)),
        ],
        out_specs=pl.BlockSpec((BR, D), lambda i: (i, 0)),
        out_shape=jax.ShapeDtypeStruct((N, D), jnp.float32),
    )(pp, hs, dinv, b, w)


def _tc3_body(pp_ref, hs_ref, dinv_ref, b_ref, wfc_ref, bfc_ref, out_ref):
    dinv = dinv_ref[...][:, :1]
    h = jnp.maximum(
        (pp_ref[0] + pp_ref[1] + hs_ref[...]) * dinv + b_ref[...], 0.0)
    out_ref[...] = jnp.dot(
        h, wfc_ref[...], preferred_element_type=jnp.float32) + bfc_ref[...]


def _tc3(pp, hs, dinv, b, wfc, bfc):
    return pl.pallas_call(
        _tc3_body,
        grid=(N // BR,),
        in_specs=[
            pl.BlockSpec((NC, BR, D), lambda i: (0, i, 0)),
            pl.BlockSpec((BR, D), lambda i: (i, 0)),
            pl.BlockSpec((BR, DW), lambda i: (i, 0)),
            pl.BlockSpec((1, D), lambda i: (0, 0)),
            pl.BlockSpec((D, D), lambda i: (0, 0)),
            pl.BlockSpec((1, D), lambda i: (0, 0)),
        ],
        out_specs=pl.BlockSpec((BR, D), lambda i: (i, 0)),
        out_shape=jax.ShapeDtypeStruct((N, D), jnp.float32),
    )(pp, hs, dinv, b, wfc, bfc)


def kernel(x, edge_index, batch, W1, b1, W2, b2, Wfc, bfc):
    ei = edge_index.astype(jnp.int32)
    ept_real = E // NW
    zeros_rows = jnp.zeros((ZR, D), jnp.float32)
    ones_k = jnp.ones((K, D), jnp.float32)

    padd = EPT - ept_real
    padd_dst = N + (jnp.arange(padd, dtype=jnp.int32)[None, :]
                    + (jnp.arange(NW, dtype=jnp.int32)[:, None] % NS) * 15
                    ) % padd
    dst2 = jnp.concatenate(
        [ei[1].reshape(NW, ept_real), padd_dst], axis=1).reshape(NW * CH, K)

    pads = EPTS - ept_real
    pads_dst = N + (jnp.arange(pads, dtype=jnp.int32)[None, :]
                    + (jnp.arange(NW, dtype=jnp.int32)[:, None] % NS) * 12
                    ) % pads
    pads_src = ((pads_dst - N) * 37 + 13) % N
    src1 = jnp.concatenate(
        [ei[0].reshape(NW, ept_real), pads_src], axis=1).reshape(EPADS)
    dst1 = jnp.concatenate(
        [ei[1].reshape(NW, ept_real), pads_dst], axis=1).reshape(EPADS)

    degp = _deg(dst2, ones_k, zeros_rows).reshape(NC, NP, D)
    hs1, dinv = _tc1(degp, x, W1)
    p1 = _scatter(hs1, src1, dst1, zeros_rows).reshape(NC, NP, D)
    hs2 = _tc2(p1, hs1, dinv, b1.reshape(1, D), W2)
    p2 = _scatter(hs2, src1, dst1, zeros_rows).reshape(NC, NP, D)
    out = _tc3(p2, hs2, dinv, b2.reshape(1, D), Wfc, bfc.reshape(1, D))
    return out


# speedup vs baseline: 1.0470x; 1.0470x over previous
import functools

import jax
import jax.numpy as jnp
from jax import lax
from jax.experimental import pallas as pl
from jax.experimental.pallas import tpu as pltpu
from jax.experimental.pallas import tpu_sc as plsc

N = 10000
E = 320000
D = 128

NC = 2
NS = 16
NW = NC * NS
K = 128
EPT = 10240
CH = EPT // K
CH2 = CH // 2
EPAD = NW * EPT
NP = 10240
RPS = NP // NS
ZR = 128
DW = 16

_mesh = plsc.VectorSubcoreMesh(core_axis_name="c", subcore_axis_name="s")


def _deg_body(dst2, ones_h, zeros_h, out, dall, onesv, zbuf, acc):
    c = lax.axis_index("c")
    s = lax.axis_index("s")
    wid = c * NS + s
    pltpu.sync_copy(zeros_h, zbuf)
    for j in range(RPS // ZR):
        pltpu.sync_copy(zbuf, acc.at[pl.ds(s * RPS + j * ZR, ZR)])
    pltpu.sync_copy(ones_h, onesv)
    pltpu.sync_copy(dst2.at[pl.ds(wid * CH, CH)], dall)
    plsc.subcore_barrier()

    def step(i, carry):
        pltpu.sync_copy(onesv, acc.at[dall.at[i]], add=True)
        return carry

    lax.fori_loop(0, CH, step, 0)
    plsc.subcore_barrier()
    for j in range(RPS // ZR):
        r0 = s * RPS + j * ZR
        pltpu.sync_copy(acc.at[pl.ds(r0, ZR)], zbuf)
        pltpu.sync_copy(zbuf, out.at[pl.ds(c * NP + r0, ZR)])


_deg = functools.partial(
    pl.kernel,
    out_type=jax.ShapeDtypeStruct((NC * NP, D), jnp.float32),
    mesh=_mesh,
    scratch_types=[
        pltpu.VMEM((CH, K), jnp.int32),
        pltpu.VMEM((K, D), jnp.float32),
        pltpu.VMEM((ZR, D), jnp.float32),
        pltpu.VMEM_SHARED((NP, D), jnp.float32),
    ],
)(_deg_body)


def _scatter_body(hs, src2, dst2, zeros_h, out, sall, dall, rows0, rows1,
                  acc, sem0, sem1):
    c = lax.axis_index("c")
    s = lax.axis_index("s")
    wid = c * NS + s
    pltpu.sync_copy(zeros_h, rows0)
    for j in range(RPS // ZR):
        pltpu.sync_copy(rows0, acc.at[pl.ds(s * RPS + j * ZR, ZR)])
    plsc.subcore_barrier()

    for h in range(2):
        pltpu.sync_copy(src2.at[pl.ds(wid * CH + h * CH2, CH2)], sall)
        pltpu.sync_copy(dst2.at[pl.ds(wid * CH + h * CH2, CH2)], dall)
        pltpu.async_copy(hs.at[sall.at[0]], rows0, sem0)

        def step(i2, carry):
            c0 = i2 * 2
            pltpu.async_copy(hs.at[sall.at[c0 + 1]], rows1, sem1)
            pltpu.make_async_copy(hs.at[sall.at[0]], rows0, sem0).wait()
            pltpu.sync_copy(rows0, acc.at[dall.at[c0]], add=True)

            @pl.when(i2 < CH2 // 2 - 1)
            def _():
                pltpu.async_copy(hs.at[sall.at[c0 + 2]], rows0, sem0)

            pltpu.make_async_copy(hs.at[sall.at[0]], rows1, sem1).wait()
            pltpu.sync_copy(rows1, acc.at[dall.at[c0 + 1]], add=True)
            return carry

        lax.fori_loop(0, CH2 // 2, step, 0)
    plsc.subcore_barrier()
    for j in range(RPS // ZR):
        r0 = s * RPS + j * ZR
        pltpu.sync_copy(acc.at[pl.ds(r0, ZR)], rows0)
        pltpu.sync_copy(rows0, out.at[pl.ds(c * NP + r0, ZR)])


_scatter = functools.partial(
    pl.kernel,
    out_type=jax.ShapeDtypeStruct((NC * NP, D), jnp.float32),
    mesh=_mesh,
    scratch_types=[
        pltpu.VMEM((CH2, K), jnp.int32),
        pltpu.VMEM((CH2, K), jnp.int32),
        pltpu.VMEM((K, D), jnp.float32),
        pltpu.VMEM((K, D), jnp.float32),
        pltpu.VMEM_SHARED((NP, D), jnp.float32),
        pltpu.SemaphoreType.DMA,
        pltpu.SemaphoreType.DMA,
    ],
)(_scatter_body)


BR = 1000


def _tc1_body(dp_ref, x_ref, w_ref, hs_ref, dinv_ref):
    deg = 1.0 + dp_ref[0][:, :1] + dp_ref[1][:, :1]
    dinv = lax.rsqrt(deg)
    h = jnp.dot(x_ref[...], w_ref[...], preferred_element_type=jnp.float32)
    hs_ref[...] = h * dinv
    dinv_ref[...] = jnp.broadcast_to(dinv, (BR, DW))


def _tc1(degp, x, w):
    return pl.pallas_call(
        _tc1_body,
        grid=(N // BR,),
        in_specs=[
            pl.BlockSpec((NC, BR, D), lambda i: (0, i, 0)),
            pl.BlockSpec((BR, D), lambda i: (i, 0)),
            pl.BlockSpec((D, D), lambda i: (0, 0)),
        ],
        out_specs=[
            pl.BlockSpec((BR, D), lambda i: (i, 0)),
            pl.BlockSpec((BR, DW), lambda i: (i, 0)),
        ],
        out_shape=[
            jax.ShapeDtypeStruct((N, D), jnp.float32),
            jax.ShapeDtypeStruct((N, DW), jnp.float32),
        ],
    )(degp, x, w)


def _tc2_body(pp_ref, hs_ref, dinv_ref, b_ref, w_ref, out_ref):
    dinv = dinv_ref[...][:, :1]
    a = (pp_ref[0] + pp_ref[1] + hs_ref[...]) * dinv + b_ref[...]
    h = jnp.maximum(a, 0.0)
    out_ref[...] = jnp.dot(
        h, w_ref[...], preferred_element_type=jnp.float32) * dinv


def _tc2(pp, hs, dinv, b, w):
    return pl.pallas_call(
        _tc2_body,
        grid=(N // BR,),
        in_specs=[
            pl.BlockSpec((NC, BR, D), lambda i: (0, i, 0)),
            pl.BlockSpec((BR, D), lambda i: (i, 0)),
            pl.BlockSpec((BR, DW), lambda i: (i, 0)),
            pl.BlockSpec((1, D), lambda i: (0, 0)),
            pl.BlockSpec((D, D), lambda i: (0, 0)),
        ],
        out_specs=pl.BlockSpec((BR, D), lambda i: (i, 0)),
        out_shape=jax.ShapeDtypeStruct((N, D), jnp.float32),
    )(pp, hs, dinv, b, w)


def _tc3_body(pp_ref, hs_ref, dinv_ref, b_ref, wfc_ref, bfc_ref, out_ref):
    dinv = dinv_ref[...][:, :1]
    h = jnp.maximum(
        (pp_ref[0] + pp_ref[1] + hs_ref[...]) * dinv + b_ref[...], 0.0)
    out_ref[...] = jnp.dot(
        h, wfc_ref[...], preferred_element_type=jnp.float32) + bfc_ref[...]


def _tc3(pp, hs, dinv, b, wfc, bfc):
    return pl.pallas_call(
        _tc3_body,
        grid=(N // BR,),
        in_specs=[
            pl.BlockSpec((NC, BR, D), lambda i: (0, i, 0)),
            pl.BlockSpec((BR, D), lambda i: (i, 0)),
            pl.BlockSpec((BR, DW), lambda i: (i, 0)),
            pl.BlockSpec((1, D), lambda i: (0, 0)),
            pl.BlockSpec((D, D), lambda i: (0, 0)),
            pl.BlockSpec((1, D), lambda i: (0, 0)),
        ],
        out_specs=pl.BlockSpec((BR, D), lambda i: (i, 0)),
        out_shape=jax.ShapeDtypeStruct((N, D), jnp.float32),
    )(pp, hs, dinv, b, wfc, bfc)


def kernel(x, edge_index, batch, W1, b1, W2, b2, Wfc, bfc):
    ei = edge_index.astype(jnp.int32)
    ept_real = E // NW
    pad = EPT - ept_real
    pad_dst = N + (jnp.arange(pad, dtype=jnp.int32)[None, :]
                   + (jnp.arange(NW, dtype=jnp.int32)[:, None] % NS) * 15
                   ) % pad
    pad_src = ((pad_dst - N) * 37 + 13) % N
    src2 = jnp.concatenate(
        [ei[0].reshape(NW, ept_real), pad_src], axis=1).reshape(NW * CH, K)
    dst2 = jnp.concatenate(
        [ei[1].reshape(NW, ept_real), pad_dst], axis=1).reshape(NW * CH, K)
    zeros_rows = jnp.zeros((ZR, D), jnp.float32)
    ones_k = jnp.ones((K, D), jnp.float32)

    degp = _deg(dst2, ones_k, zeros_rows).reshape(NC, NP, D)
    hs1, dinv = _tc1(degp, x, W1)
    p1 = _scatter(hs1, src2, dst2, zeros_rows).reshape(NC, NP, D)
    hs2 = _tc2(p1, hs1, dinv, b1.reshape(1, D), W2)
    p2 = _scatter(hs2, src2, dst2, zeros_rows).reshape(NC, NP, D)
    out = _tc3(p2, hs2, dinv, b2.reshape(1, D), Wfc, bfc.reshape(1, D))
    return out
